# Initial kernel scaffold; baseline (speedup 1.0000x reference)
import jax
import jax.numpy as jnp
from jax.experimental import pallas as pl


def kernel(x_user, x_item, ei_user_item, ei_item_user, W1_ui_root, W1_ui_nbr, b1_ui, W1_iu_root, W1_iu_nbr, b1_iu, W2_ui_root, W2_ui_nbr, b2_ui, W2_iu_root, W2_iu_nbr, b2_iu):
    raise NotImplementedError("write your pallas kernel here")


# speedup vs baseline: 4.0883x; 4.0883x over previous
import functools

import jax
import jax.numpy as jnp
from jax import lax
from jax.experimental import pallas as pl
from jax.experimental.pallas import tpu as pltpu
from jax.experimental.pallas import tpu_sc as plsc

N = 10000
D = 128
NT = 16
CH = 128
G = 32
GPT = 5
CPT = G * GPT
NCHUNK = CPT * NT
E_PAD = NCHUNK * CH
NPAD = 10240
STRIPE = NPAD // NT


def _sc_agg_body(with_counts, *refs):
    if with_counts:
        (x0, x1, src0, dst0, src1, dst1, z2, z1,
         agg0, agg1, cnt0, cnt1,
         agg_sh, cnt_sh, src_v, dst_v, rows_v, ones_v, sem) = refs
    else:
        (x0, x1, src0, dst0, src1, dst1, z2,
         agg0, agg1,
         agg_sh, src_v, dst_v, rows_v, sem) = refs

    c = lax.axis_index("c")
    s = lax.axis_index("s")
    row0 = s * STRIPE

    pltpu.sync_copy(z2.at[pl.ds(row0, STRIPE)], agg_sh.at[pl.ds(row0, STRIPE)])
    if with_counts:
        pltpu.sync_copy(z1.at[pl.ds(row0, STRIPE)],
                        cnt_sh.at[pl.ds(row0, STRIPE)])
        for i in range(CH // 16):
            ones_v[pl.ds(i * 16, 16)] = jnp.full((16,), 1.0, jnp.float32)
    plsc.subcore_barrier()

    def run_relation(x_hbm, src_hbm, dst_hbm, agg_out, cnt_out):
        def group(g, carry):
            base = s * CPT + g * G
            pltpu.sync_copy(src_hbm.at[pl.ds(base, G)], src_v)
            pltpu.sync_copy(dst_hbm.at[pl.ds(base, G)], dst_v)

            def chunk(j, carry2):
                pltpu.async_copy(x_hbm.at[src_v.at[j]], rows_v, sem).wait()
                pltpu.sync_copy(rows_v, agg_sh.at[dst_v.at[j]], add=True)
                if with_counts:
                    pltpu.sync_copy(ones_v, cnt_sh.at[dst_v.at[j]], add=True)
                return carry2

            return lax.fori_loop(0, G, chunk, carry)

        lax.fori_loop(0, GPT, group, 0)
        plsc.subcore_barrier()
        pltpu.sync_copy(agg_sh.at[pl.ds(row0, STRIPE)],
                        agg_out.at[pl.ds(row0, STRIPE)])
        if with_counts:
            pltpu.sync_copy(cnt_sh.at[pl.ds(row0, STRIPE)],
                            cnt_out.at[pl.ds(row0, STRIPE)])

    @pl.when(c == 0)
    def _():
        run_relation(x0, src0, dst0, agg0, cnt0 if with_counts else None)

    @pl.when(c == 1)
    def _():
        run_relation(x1, src1, dst1, agg1, cnt1 if with_counts else None)


@functools.lru_cache(maxsize=None)
def _make_sc_agg(with_counts):
    n_out = 4 if with_counts else 2
    out_type = tuple(
        [jax.ShapeDtypeStruct((NPAD, D), jnp.float32)] * 2
        + [jax.ShapeDtypeStruct((NPAD,), jnp.float32)] * (n_out - 2))
    scratch = [pltpu.VMEM_SHARED((NPAD, D), jnp.float32)]
    if with_counts:
        scratch.append(pltpu.VMEM_SHARED((NPAD,), jnp.float32))
    scratch += [
        pltpu.VMEM((G, CH), jnp.int32),
        pltpu.VMEM((G, CH), jnp.int32),
        pltpu.VMEM((CH, D), jnp.float32),
    ]
    if with_counts:
        scratch.append(pltpu.VMEM((CH,), jnp.float32))
    scratch.append(pltpu.SemaphoreType.DMA)
    mesh = plsc.VectorSubcoreMesh(core_axis_name="c", subcore_axis_name="s",
                                  num_cores=2, num_subcores=NT)
    return pl.kernel(
        functools.partial(_sc_agg_body, with_counts),
        out_type=out_type, mesh=mesh, scratch_types=scratch,
        name="sc_segsum_counts" if with_counts else "sc_segsum")


_BLK = 1000


def _tc1_body(x_i, agg_i, cnt_i, x_u, agg_u, cnt_u,
              wir, win, bi, wur, wun, bu,
              h_i, h_u, inv_i, inv_u):
    dn = (((1,), (1,)), ((), ()))
    ii = 1.0 / jnp.maximum(cnt_i[...], 1.0)
    iu = 1.0 / jnp.maximum(cnt_u[...], 1.0)
    inv_i[...] = ii
    inv_u[...] = iu
    hi = (lax.dot_general(x_i[...], wir[...], dn,
                          preferred_element_type=jnp.float32)
          + lax.dot_general(agg_i[...] * ii, win[...], dn,
                            preferred_element_type=jnp.float32)
          + bi[...])
    hu = (lax.dot_general(x_u[...], wur[...], dn,
                          preferred_element_type=jnp.float32)
          + lax.dot_general(agg_u[...] * iu, wun[...], dn,
                            preferred_element_type=jnp.float32)
          + bu[...])
    h_i[...] = jnp.maximum(hi, 0.0)
    h_u[...] = jnp.maximum(hu, 0.0)


def _tc2_body(x_i, agg_i, inv_i, x_u, agg_u, inv_u,
              wir, win, bi, wur, wun, bu,
              o_i, o_u):
    dn = (((1,), (1,)), ((), ()))
    o_i[...] = (lax.dot_general(x_i[...], wir[...], dn,
                                preferred_element_type=jnp.float32)
                + lax.dot_general(agg_i[...] * inv_i[...], win[...], dn,
                                  preferred_element_type=jnp.float32)
                + bi[...])
    o_u[...] = (lax.dot_general(x_u[...], wur[...], dn,
                                preferred_element_type=jnp.float32)
                + lax.dot_general(agg_u[...] * inv_u[...], wun[...], dn,
                                  preferred_element_type=jnp.float32)
                + bu[...])


def _feat_spec():
    return pl.BlockSpec((_BLK, D), lambda i: (i, 0))


def _col_spec():
    return pl.BlockSpec((_BLK, 1), lambda i: (i, 0))


def _w_spec():
    return pl.BlockSpec((D, D), lambda i: (0, 0))


def _b_spec():
    return pl.BlockSpec((1, D), lambda i: (0, 0))


_tc1 = pl.pallas_call(
    _tc1_body,
    grid=(N // _BLK,),
    in_specs=[_feat_spec(), _feat_spec(), _col_spec(),
              _feat_spec(), _feat_spec(), _col_spec(),
              _w_spec(), _w_spec(), _b_spec(),
              _w_spec(), _w_spec(), _b_spec()],
    out_specs=[_feat_spec(), _feat_spec(), _col_spec(), _col_spec()],
    out_shape=[jax.ShapeDtypeStruct((N, D), jnp.float32),
               jax.ShapeDtypeStruct((N, D), jnp.float32),
               jax.ShapeDtypeStruct((N, 1), jnp.float32),
               jax.ShapeDtypeStruct((N, 1), jnp.float32)],
)

_tc2 = pl.pallas_call(
    _tc2_body,
    grid=(N // _BLK,),
    in_specs=[_feat_spec(), _feat_spec(), _col_spec(),
              _feat_spec(), _feat_spec(), _col_spec(),
              _w_spec(), _w_spec(), _b_spec(),
              _w_spec(), _w_spec(), _b_spec()],
    out_specs=[_feat_spec(), _feat_spec()],
    out_shape=[jax.ShapeDtypeStruct((N, D), jnp.float32),
               jax.ShapeDtypeStruct((N, D), jnp.float32)],
)


def _prep_edges(ei):
    e = ei.shape[1]
    pad = E_PAD - e
    src = jnp.concatenate([ei[0], jnp.zeros((pad,), jnp.int32)])
    dst = jnp.concatenate([ei[1], jnp.full((pad,), N, jnp.int32)])
    return src.reshape(NCHUNK, CH), dst.reshape(NCHUNK, CH)


def kernel(x_user, x_item, ei_user_item, ei_item_user,
           W1_ui_root, W1_ui_nbr, b1_ui, W1_iu_root, W1_iu_nbr, b1_iu,
           W2_ui_root, W2_ui_nbr, b2_ui, W2_iu_root, W2_iu_nbr, b2_iu):
    src_ui, dst_ui = _prep_edges(ei_user_item)
    src_iu, dst_iu = _prep_edges(ei_item_user)
    z2 = jnp.zeros((NPAD, D), jnp.float32)
    z1 = jnp.zeros((NPAD,), jnp.float32)

    agg1_i, agg1_u, cnt_i, cnt_u = _make_sc_agg(True)(
        x_user, x_item, src_ui, dst_ui, src_iu, dst_iu, z2, z1)

    h_i, h_u, inv_i, inv_u = _tc1(
        x_item, agg1_i[:N], cnt_i[:N].reshape(N, 1),
        x_user, agg1_u[:N], cnt_u[:N].reshape(N, 1),
        W1_ui_root, W1_ui_nbr, b1_ui.reshape(1, D),
        W1_iu_root, W1_iu_nbr, b1_iu.reshape(1, D))

    agg2_i, agg2_u = _make_sc_agg(False)(
        h_u, h_i, src_ui, dst_ui, src_iu, dst_iu, z2)

    out_i, out_u = _tc2(
        h_i, agg2_i[:N], inv_i, h_u, agg2_u[:N], inv_u,
        W2_ui_root, W2_ui_nbr, b2_ui.reshape(1, D),
        W2_iu_root, W2_iu_nbr, b2_iu.reshape(1, D))

    return (out_u, out_i)


# speedup vs baseline: 5.0122x; 1.2260x over previous
import functools

import jax
import jax.numpy as jnp
from jax import lax
from jax.experimental import pallas as pl
from jax.experimental.pallas import tpu as pltpu
from jax.experimental.pallas import tpu_sc as plsc

N = 10000
D = 128
NT = 16
CH = 64
CPT = 320
NCHUNK = CPT * NT
E_PAD = NCHUNK * CH
NPAD = 10240
STRIPE = NPAD // NT
K = 4
GL = 2
GIDX = 40
NGRP = CPT // GIDX


def _sc_agg_body(with_counts, *refs):
    if with_counts:
        (x0, x1, src0, dst0, src1, dst1, z2, z1,
         agg0, agg1, cnt0, cnt1,
         agg_sh, cnt_sh, src_v, dst_v, rows_v, ones_v,
         sem_c, *slot_sems) = refs
    else:
        (x0, x1, src0, dst0, src1, dst1, z2,
         agg0, agg1,
         agg_sh, src_v, dst_v, rows_v, *slot_sems) = refs
    sem_g = slot_sems[:K]
    sem_s = slot_sems[K:2 * K]

    c = lax.axis_index("c")
    s = lax.axis_index("s")
    row0 = s * STRIPE

    pltpu.sync_copy(z2.at[pl.ds(row0, STRIPE)], agg_sh.at[pl.ds(row0, STRIPE)])
    if with_counts:
        pltpu.sync_copy(z1.at[pl.ds(row0, STRIPE)],
                        cnt_sh.at[pl.ds(row0, STRIPE)])
        for i in range(CH // 16):
            ones_v[pl.ds(i * 16, 16)] = jnp.full((16,), 1.0, jnp.float32)
    plsc.subcore_barrier()

    def run_relation(x_hbm, src_hbm, dst_hbm, agg_out, cnt_out):
        def gather(j, k):
            pltpu.async_copy(x_hbm.at[src_v.at[j]], rows_v.at[k], sem_g[k])

        def wait_gather(k):
            pltpu.make_async_copy(
                x_hbm.at[src_v.at[0]], rows_v.at[k], sem_g[k]).wait()

        def scatter(j, k):
            pltpu.async_copy(rows_v.at[k], agg_sh.at[dst_v.at[j]],
                             sem_s[k], add=True)
            if with_counts:
                pltpu.async_copy(ones_v, cnt_sh.at[dst_v.at[j]],
                                 sem_c, add=True)

        def wait_scatter(k):
            pltpu.make_async_copy(
                rows_v.at[k], agg_sh.at[dst_v.at[0]], sem_s[k]).wait()

        def wait_count():
            pltpu.make_async_copy(
                ones_v, cnt_sh.at[dst_v.at[0]], sem_c).wait()

        def group(g, carry):
            base = s * CPT + g * GIDX
            pltpu.sync_copy(src_hbm.at[pl.ds(base, GIDX)], src_v)
            pltpu.sync_copy(dst_hbm.at[pl.ds(base, GIDX)], dst_v)

            for k in range(GL):
                gather(k, k)

            def step(it, carry2):
                jb = it * K
                for k in range(K):
                    j = jb + k
                    jn = j + GL
                    kn = (k + GL) % K

                    @pl.when(jnp.logical_and(jn < GIDX, jn >= K))
                    def _():
                        wait_scatter(kn)

                    @pl.when(jn < GIDX)
                    def _():
                        gather(jn, kn)

                    wait_gather(k)
                    scatter(j, k)
                    if with_counts:
                        @pl.when(j >= K)
                        def _():
                            wait_count()
                return carry2

            lax.fori_loop(0, GIDX // K, step, 0)
            for k in range(K):
                wait_scatter((GIDX - K + k) % K)
                if with_counts:
                    wait_count()
            return carry

        lax.fori_loop(0, NGRP, group, 0)
        plsc.subcore_barrier()
        pltpu.sync_copy(agg_sh.at[pl.ds(row0, STRIPE)],
                        agg_out.at[pl.ds(row0, STRIPE)])
        if with_counts:
            pltpu.sync_copy(cnt_sh.at[pl.ds(row0, STRIPE)],
                            cnt_out.at[pl.ds(row0, STRIPE)])

    @pl.when(c == 0)
    def _():
        run_relation(x0, src0, dst0, agg0, cnt0 if with_counts else None)

    @pl.when(c == 1)
    def _():
        run_relation(x1, src1, dst1, agg1, cnt1 if with_counts else None)


@functools.lru_cache(maxsize=None)
def _make_sc_agg(with_counts):
    n_out = 4 if with_counts else 2
    out_type = tuple(
        [jax.ShapeDtypeStruct((NPAD, D), jnp.float32)] * 2
        + [jax.ShapeDtypeStruct((NPAD,), jnp.float32)] * (n_out - 2))
    scratch = [pltpu.VMEM_SHARED((NPAD, D), jnp.float32)]
    if with_counts:
        scratch.append(pltpu.VMEM_SHARED((NPAD,), jnp.float32))
    scratch += [
        pltpu.VMEM((GIDX, CH), jnp.int32),
        pltpu.VMEM((GIDX, CH), jnp.int32),
        pltpu.VMEM((K, CH, D), jnp.float32),
    ]
    if with_counts:
        scratch.append(pltpu.VMEM((CH,), jnp.float32))
        scratch.append(pltpu.SemaphoreType.DMA)
    scratch += [pltpu.SemaphoreType.DMA] * (2 * K)
    mesh = plsc.VectorSubcoreMesh(core_axis_name="c", subcore_axis_name="s",
                                  num_cores=2, num_subcores=NT)
    return pl.kernel(
        functools.partial(_sc_agg_body, with_counts),
        out_type=out_type, mesh=mesh, scratch_types=scratch,
        name="sc_segsum_counts" if with_counts else "sc_segsum")


_BLK = 1000


def _tc1_body(x_i, agg_i, cnt_i, x_u, agg_u, cnt_u,
              wir, win, bi, wur, wun, bu,
              h_i, h_u, inv_i, inv_u):
    dn = (((1,), (1,)), ((), ()))
    ii = 1.0 / jnp.maximum(cnt_i[...], 1.0)
    iu = 1.0 / jnp.maximum(cnt_u[...], 1.0)
    inv_i[...] = ii
    inv_u[...] = iu
    hi = (lax.dot_general(x_i[...], wir[...], dn,
                          preferred_element_type=jnp.float32)
          + lax.dot_general(agg_i[...] * ii, win[...], dn,
                            preferred_element_type=jnp.float32)
          + bi[...])
    hu = (lax.dot_general(x_u[...], wur[...], dn,
                          preferred_element_type=jnp.float32)
          + lax.dot_general(agg_u[...] * iu, wun[...], dn,
                            preferred_element_type=jnp.float32)
          + bu[...])
    h_i[...] = jnp.maximum(hi, 0.0)
    h_u[...] = jnp.maximum(hu, 0.0)


def _tc2_body(x_i, agg_i, inv_i, x_u, agg_u, inv_u,
              wir, win, bi, wur, wun, bu,
              o_i, o_u):
    dn = (((1,), (1,)), ((), ()))
    o_i[...] = (lax.dot_general(x_i[...], wir[...], dn,
                                preferred_element_type=jnp.float32)
                + lax.dot_general(agg_i[...] * inv_i[...], win[...], dn,
                                  preferred_element_type=jnp.float32)
                + bi[...])
    o_u[...] = (lax.dot_general(x_u[...], wur[...], dn,
                                preferred_element_type=jnp.float32)
                + lax.dot_general(agg_u[...] * inv_u[...], wun[...], dn,
                                  preferred_element_type=jnp.float32)
                + bu[...])


def _feat_spec():
    return pl.BlockSpec((_BLK, D), lambda i: (i, 0))


def _col_spec():
    return pl.BlockSpec((_BLK, 1), lambda i: (i, 0))


def _w_spec():
    return pl.BlockSpec((D, D), lambda i: (0, 0))


def _b_spec():
    return pl.BlockSpec((1, D), lambda i: (0, 0))


_tc1 = pl.pallas_call(
    _tc1_body,
    grid=(N // _BLK,),
    in_specs=[_feat_spec(), _feat_spec(), _col_spec(),
              _feat_spec(), _feat_spec(), _col_spec(),
              _w_spec(), _w_spec(), _b_spec(),
              _w_spec(), _w_spec(), _b_spec()],
    out_specs=[_feat_spec(), _feat_spec(), _col_spec(), _col_spec()],
    out_shape=[jax.ShapeDtypeStruct((N, D), jnp.float32),
               jax.ShapeDtypeStruct((N, D), jnp.float32),
               jax.ShapeDtypeStruct((N, 1), jnp.float32),
               jax.ShapeDtypeStruct((N, 1), jnp.float32)],
)

_tc2 = pl.pallas_call(
    _tc2_body,
    grid=(N // _BLK,),
    in_specs=[_feat_spec(), _feat_spec(), _col_spec(),
              _feat_spec(), _feat_spec(), _col_spec(),
              _w_spec(), _w_spec(), _b_spec(),
              _w_spec(), _w_spec(), _b_spec()],
    out_specs=[_feat_spec(), _feat_spec()],
    out_shape=[jax.ShapeDtypeStruct((N, D), jnp.float32),
               jax.ShapeDtypeStruct((N, D), jnp.float32)],
)


def _prep_edges(ei):
    e = ei.shape[1]
    pad = E_PAD - e
    src = jnp.concatenate([ei[0], jnp.zeros((pad,), jnp.int32)])
    dst = jnp.concatenate([ei[1], jnp.full((pad,), N, jnp.int32)])
    return src.reshape(NCHUNK, CH), dst.reshape(NCHUNK, CH)


def kernel(x_user, x_item, ei_user_item, ei_item_user,
           W1_ui_root, W1_ui_nbr, b1_ui, W1_iu_root, W1_iu_nbr, b1_iu,
           W2_ui_root, W2_ui_nbr, b2_ui, W2_iu_root, W2_iu_nbr, b2_iu):
    src_ui, dst_ui = _prep_edges(ei_user_item)
    src_iu, dst_iu = _prep_edges(ei_item_user)
    z2 = jnp.zeros((NPAD, D), jnp.float32)
    z1 = jnp.zeros((NPAD,), jnp.float32)

    agg1_i, agg1_u, cnt_i, cnt_u = _make_sc_agg(True)(
        x_user, x_item, src_ui, dst_ui, src_iu, dst_iu, z2, z1)

    h_i, h_u, inv_i, inv_u = _tc1(
        x_item, agg1_i[:N], cnt_i[:N].reshape(N, 1),
        x_user, agg1_u[:N], cnt_u[:N].reshape(N, 1),
        W1_ui_root, W1_ui_nbr, b1_ui.reshape(1, D),
        W1_iu_root, W1_iu_nbr, b1_iu.reshape(1, D))

    agg2_i, agg2_u = _make_sc_agg(False)(
        h_u, h_i, src_ui, dst_ui, src_iu, dst_iu, z2)

    out_i, out_u = _tc2(
        h_i, agg2_i[:N], inv_i, h_u, agg2_u[:N], inv_u,
        W2_ui_root, W2_ui_nbr, b2_ui.reshape(1, D),
        W2_iu_root, W2_iu_nbr, b2_iu.reshape(1, D))

    return (out_u, out_i)


# speedup vs baseline: 5.0333x; 1.0042x over previous
import functools

import jax
import jax.numpy as jnp
from jax import lax
from jax.experimental import pallas as pl
from jax.experimental.pallas import tpu as pltpu
from jax.experimental.pallas import tpu_sc as plsc

N = 10000
D = 128
NT = 16
CH = 64
CPT = 320
NCHUNK = CPT * NT
E_PAD = NCHUNK * CH
NPAD = 10240
STRIPE = NPAD // NT
K = 4
GL = 3
GIDX = 40
NGRP = CPT // GIDX


def _sc_agg_body(with_counts, *refs):
    if with_counts:
        (x0, x1, src0, dst0, src1, dst1, z2, z1,
         agg0, agg1, cnt0, cnt1,
         agg_sh, cnt_sh, src_v, dst_v, rows_v, ones_v,
         sem_c, *slot_sems) = refs
    else:
        (x0, x1, src0, dst0, src1, dst1, z2,
         agg0, agg1,
         agg_sh, src_v, dst_v, rows_v, *slot_sems) = refs
    sem_g = slot_sems[:K]
    sem_s = slot_sems[K:2 * K]

    c = lax.axis_index("c")
    s = lax.axis_index("s")
    row0 = s * STRIPE

    pltpu.sync_copy(z2.at[pl.ds(row0, STRIPE)], agg_sh.at[pl.ds(row0, STRIPE)])
    if with_counts:
        pltpu.sync_copy(z1.at[pl.ds(row0, STRIPE)],
                        cnt_sh.at[pl.ds(row0, STRIPE)])
        for i in range(CH // 16):
            ones_v[pl.ds(i * 16, 16)] = jnp.full((16,), 1.0, jnp.float32)
    plsc.subcore_barrier()

    def run_relation(x_hbm, src_hbm, dst_hbm, agg_out, cnt_out):
        def gather(j, k):
            pltpu.async_copy(x_hbm.at[src_v.at[j]], rows_v.at[k], sem_g[k])

        def wait_gather(k):
            pltpu.make_async_copy(
                x_hbm.at[src_v.at[0]], rows_v.at[k], sem_g[k]).wait()

        def scatter(j, k):
            pltpu.async_copy(rows_v.at[k], agg_sh.at[dst_v.at[j]],
                             sem_s[k], add=True)
            if with_counts:
                pltpu.async_copy(ones_v, cnt_sh.at[dst_v.at[j]],
                                 sem_c, add=True)

        def wait_scatter(k):
            pltpu.make_async_copy(
                rows_v.at[k], agg_sh.at[dst_v.at[0]], sem_s[k]).wait()

        def wait_count():
            pltpu.make_async_copy(
                ones_v, cnt_sh.at[dst_v.at[0]], sem_c).wait()

        def group(g, carry):
            base = s * CPT + g * GIDX
            pltpu.sync_copy(src_hbm.at[pl.ds(base, GIDX)], src_v)
            pltpu.sync_copy(dst_hbm.at[pl.ds(base, GIDX)], dst_v)

            for k in range(GL):
                gather(k, k)

            def step(it, carry2):
                jb = it * K
                for k in range(K):
                    j = jb + k
                    jn = j + GL
                    kn = (k + GL) % K

                    @pl.when(jnp.logical_and(jn < GIDX, jn >= K))
                    def _():
                        wait_scatter(kn)

                    @pl.when(jn < GIDX)
                    def _():
                        gather(jn, kn)

                    wait_gather(k)
                    scatter(j, k)
                    if with_counts:
                        @pl.when(j >= K)
                        def _():
                            wait_count()
                return carry2

            lax.fori_loop(0, GIDX // K, step, 0)
            for k in range(K):
                wait_scatter((GIDX - K + k) % K)
                if with_counts:
                    wait_count()
            return carry

        lax.fori_loop(0, NGRP, group, 0)
        plsc.subcore_barrier()
        pltpu.sync_copy(agg_sh.at[pl.ds(row0, STRIPE)],
                        agg_out.at[pl.ds(row0, STRIPE)])
        if with_counts:
            pltpu.sync_copy(cnt_sh.at[pl.ds(row0, STRIPE)],
                            cnt_out.at[pl.ds(row0, STRIPE)])

    @pl.when(c == 0)
    def _():
        run_relation(x0, src0, dst0, agg0, cnt0 if with_counts else None)

    @pl.when(c == 1)
    def _():
        run_relation(x1, src1, dst1, agg1, cnt1 if with_counts else None)


@functools.lru_cache(maxsize=None)
def _make_sc_agg(with_counts):
    n_out = 4 if with_counts else 2
    out_type = tuple(
        [jax.ShapeDtypeStruct((NPAD, D), jnp.float32)] * 2
        + [jax.ShapeDtypeStruct((NPAD,), jnp.float32)] * (n_out - 2))
    scratch = [pltpu.VMEM_SHARED((NPAD, D), jnp.float32)]
    if with_counts:
        scratch.append(pltpu.VMEM_SHARED((NPAD,), jnp.float32))
    scratch += [
        pltpu.VMEM((GIDX, CH), jnp.int32),
        pltpu.VMEM((GIDX, CH), jnp.int32),
        pltpu.VMEM((K, CH, D), jnp.float32),
    ]
    if with_counts:
        scratch.append(pltpu.VMEM((CH,), jnp.float32))
        scratch.append(pltpu.SemaphoreType.DMA)
    scratch += [pltpu.SemaphoreType.DMA] * (2 * K)
    mesh = plsc.VectorSubcoreMesh(core_axis_name="c", subcore_axis_name="s",
                                  num_cores=2, num_subcores=NT)
    return pl.kernel(
        functools.partial(_sc_agg_body, with_counts),
        out_type=out_type, mesh=mesh, scratch_types=scratch,
        name="sc_segsum_counts" if with_counts else "sc_segsum")


_BLK = 1000


def _tc1_body(x_i, agg_i, cnt_i, x_u, agg_u, cnt_u,
              wir, win, bi, wur, wun, bu,
              h_i, h_u, inv_i, inv_u):
    dn = (((1,), (1,)), ((), ()))
    ii = 1.0 / jnp.maximum(cnt_i[...], 1.0)
    iu = 1.0 / jnp.maximum(cnt_u[...], 1.0)
    inv_i[...] = ii
    inv_u[...] = iu
    hi = (lax.dot_general(x_i[...], wir[...], dn,
                          preferred_element_type=jnp.float32)
          + lax.dot_general(agg_i[...] * ii, win[...], dn,
                            preferred_element_type=jnp.float32)
          + bi[...])
    hu = (lax.dot_general(x_u[...], wur[...], dn,
                          preferred_element_type=jnp.float32)
          + lax.dot_general(agg_u[...] * iu, wun[...], dn,
                            preferred_element_type=jnp.float32)
          + bu[...])
    h_i[...] = jnp.maximum(hi, 0.0)
    h_u[...] = jnp.maximum(hu, 0.0)


def _tc2_body(x_i, agg_i, inv_i, x_u, agg_u, inv_u,
              wir, win, bi, wur, wun, bu,
              o_i, o_u):
    dn = (((1,), (1,)), ((), ()))
    o_i[...] = (lax.dot_general(x_i[...], wir[...], dn,
                                preferred_element_type=jnp.float32)
                + lax.dot_general(agg_i[...] * inv_i[...], win[...], dn,
                                  preferred_element_type=jnp.float32)
                + bi[...])
    o_u[...] = (lax.dot_general(x_u[...], wur[...], dn,
                                preferred_element_type=jnp.float32)
                + lax.dot_general(agg_u[...] * inv_u[...], wun[...], dn,
                                  preferred_element_type=jnp.float32)
                + bu[...])


def _feat_spec():
    return pl.BlockSpec((_BLK, D), lambda i: (i, 0))


def _col_spec():
    return pl.BlockSpec((_BLK, 1), lambda i: (i, 0))


def _w_spec():
    return pl.BlockSpec((D, D), lambda i: (0, 0))


def _b_spec():
    return pl.BlockSpec((1, D), lambda i: (0, 0))


_tc1 = pl.pallas_call(
    _tc1_body,
    grid=(N // _BLK,),
    in_specs=[_feat_spec(), _feat_spec(), _col_spec(),
              _feat_spec(), _feat_spec(), _col_spec(),
              _w_spec(), _w_spec(), _b_spec(),
              _w_spec(), _w_spec(), _b_spec()],
    out_specs=[_feat_spec(), _feat_spec(), _col_spec(), _col_spec()],
    out_shape=[jax.ShapeDtypeStruct((N, D), jnp.float32),
               jax.ShapeDtypeStruct((N, D), jnp.float32),
               jax.ShapeDtypeStruct((N, 1), jnp.float32),
               jax.ShapeDtypeStruct((N, 1), jnp.float32)],
)

_tc2 = pl.pallas_call(
    _tc2_body,
    grid=(N // _BLK,),
    in_specs=[_feat_spec(), _feat_spec(), _col_spec(),
              _feat_spec(), _feat_spec(), _col_spec(),
              _w_spec(), _w_spec(), _b_spec(),
              _w_spec(), _w_spec(), _b_spec()],
    out_specs=[_feat_spec(), _feat_spec()],
    out_shape=[jax.ShapeDtypeStruct((N, D), jnp.float32),
               jax.ShapeDtypeStruct((N, D), jnp.float32)],
)


def _prep_edges(ei):
    e = ei.shape[1]
    pad = E_PAD - e
    src = jnp.concatenate([ei[0], jnp.zeros((pad,), jnp.int32)])
    dst = jnp.concatenate([ei[1], jnp.full((pad,), N, jnp.int32)])
    return src.reshape(NCHUNK, CH), dst.reshape(NCHUNK, CH)


def kernel(x_user, x_item, ei_user_item, ei_item_user,
           W1_ui_root, W1_ui_nbr, b1_ui, W1_iu_root, W1_iu_nbr, b1_iu,
           W2_ui_root, W2_ui_nbr, b2_ui, W2_iu_root, W2_iu_nbr, b2_iu):
    src_ui, dst_ui = _prep_edges(ei_user_item)
    src_iu, dst_iu = _prep_edges(ei_item_user)
    z2 = jnp.zeros((NPAD, D), jnp.float32)
    z1 = jnp.zeros((NPAD,), jnp.float32)

    agg1_i, agg1_u, cnt_i, cnt_u = _make_sc_agg(True)(
        x_user, x_item, src_ui, dst_ui, src_iu, dst_iu, z2, z1)

    h_i, h_u, inv_i, inv_u = _tc1(
        x_item, agg1_i[:N], cnt_i[:N].reshape(N, 1),
        x_user, agg1_u[:N], cnt_u[:N].reshape(N, 1),
        W1_ui_root, W1_ui_nbr, b1_ui.reshape(1, D),
        W1_iu_root, W1_iu_nbr, b1_iu.reshape(1, D))

    agg2_i, agg2_u = _make_sc_agg(False)(
        h_u, h_i, src_ui, dst_ui, src_iu, dst_iu, z2)

    out_i, out_u = _tc2(
        h_i, agg2_i[:N], inv_i, h_u, agg2_u[:N], inv_u,
        W2_ui_root, W2_ui_nbr, b2_ui.reshape(1, D),
        W2_iu_root, W2_iu_nbr, b2_iu.reshape(1, D))

    return (out_u, out_i)
